# baseline (speedup 1.0000x reference)
import functools

import jax
import jax.numpy as jnp
from jax import lax
from jax.experimental import pallas as pl
from jax.experimental.pallas import tpu as pltpu, tpu_sc as plsc

DECAY = 0.98
LR = 0.05
B, T, D = 4, 128, 64
M = 65536
BM = 2048
QT = B * T
L = 16
NV = T // L


def _sc_sort_body(sur_hbm, tok_hbm, ss_hbm, sur_v, tok_v, ss_v):
    wid = lax.axis_index("s") * 2 + lax.axis_index("c")

    @pl.when(wid == 0)
    def _tile0():
        pltpu.sync_copy(sur_hbm, sur_v)
        svecs = []
        for v in range(NV):
            acc = sur_v[0, pl.ds(v * L, L)]
            for b in range(1, B):
                acc = acc + sur_v[b, pl.ds(v * L, L)]
            sv = acc * 0.25
            svecs.append(sv)

        cnts = tuple(jnp.zeros((L,), jnp.int32) for _ in range(NV))
        for ub in range(NV):
            src = svecs[ub]

            def rank_step(r, cnts, ub=ub, src=src):
                su = lax.gather(
                    src, jnp.full((L, 1), r, jnp.int32),
                    dimension_numbers=lax.GatherDimensionNumbers(
                        offset_dims=(), collapsed_slice_dims=(0,),
                        start_index_map=(0,)),
                    slice_sizes=(1,),
                    mode=lax.GatherScatterMode.PROMISE_IN_BOUNDS)
                u = ub * L + r
                new = []
                for v in range(NV):
                    lane = lax.iota(jnp.int32, L) + (v * L)
                    gt = (su > svecs[v]).astype(jnp.int32)
                    tie = ((su == svecs[v]) & (u < lane)).astype(jnp.int32)
                    new.append(cnts[v] + gt + tie)
                return tuple(new)

            cnts = lax.fori_loop(0, L, rank_step, cnts)

        for v in range(NV):
            lane = lax.iota(jnp.int32, L) + (v * L)
            plsc.store_scatter(tok_v, [cnts[v]], lane)
            plsc.store_scatter(ss_v, [cnts[v]], svecs[v])
        pltpu.sync_copy(tok_v, tok_hbm)
        pltpu.sync_copy(ss_v, ss_hbm)


_sc_sort = functools.partial(
    pl.kernel,
    out_type=(jax.ShapeDtypeStruct((T,), jnp.int32),
              jax.ShapeDtypeStruct((T,), jnp.float32)),
    compiler_params=pltpu.CompilerParams(needs_layout_passes=False),
    mesh=plsc.VectorSubcoreMesh(core_axis_name="c", subcore_axis_name="s"),
    scratch_types=[
        pltpu.VMEM((B, T), jnp.float32),
        pltpu.VMEM((T,), jnp.int32),
        pltpu.VMEM((T,), jnp.float32),
    ],
)(_sc_sort_body)


def _flash_body(hf_ref, mem_ref, str_ref, tok_ref, ss_ref, out_ref,
                qn_ref, acc_ref, dmem_ref):
    j = pl.program_id(0)
    nb = pl.num_programs(0)
    ones_row = jnp.ones((1, D), jnp.float32)

    @pl.when(j == 0)
    def _prologue():
        hfv = hf_ref[...]
        qss = jnp.sum(hfv * hfv, axis=1, keepdims=True)
        qn_ref[...] = (hfv / jnp.maximum(jnp.sqrt(qss), 1e-12)
                       ).astype(jnp.bfloat16)

        c_io = jax.lax.broadcasted_iota(jnp.int32, (T, T), 1)
        ohot = (tok_ref[...] == c_io).astype(jnp.float32)

        mh = (hfv[0:T] + hfv[T:2 * T] + hfv[2 * T:3 * T] + hfv[3 * T:4 * T]) * 0.25
        mss = jnp.sum(mh * mh, axis=1, keepdims=True)
        mhn = mh / jnp.maximum(jnp.sqrt(mss), 1e-12)
        dmem_ref[...] = LR * jax.lax.dot_general(
            ohot, mhn, (((1,), (0,)), ((), ())),
            preferred_element_type=jnp.float32)

        acc_ref[...] = jnp.zeros((QT, D + 1), jnp.float32)

    mem_blk = mem_ref[...]
    memb = mem_blk.astype(jnp.bfloat16)
    sq = mem_blk * mem_blk
    nss = jax.lax.dot_general(
        ones_row, sq, (((1,), (1,)), ((), ())),
        preferred_element_type=jnp.float32)
    rowmul = (DECAY * DECAY * str_ref[...]) / jnp.maximum(
        DECAY * jnp.sqrt(nss), 1e-12)
    logits = jax.lax.dot_general(
        qn_ref[...], memb, (((1,), (1,)), ((), ())),
        preferred_element_type=jnp.float32)
    p = jnp.exp(logits * rowmul)
    pb = p.astype(jnp.bfloat16)
    vaug = jnp.concatenate(
        [memb, jnp.ones((BM, 1), jnp.bfloat16)], axis=1)
    acc_ref[...] += jax.lax.dot_general(
        pb, vaug, (((1,), (0,)), ((), ())),
        preferred_element_type=jnp.float32)

    @pl.when(j == 0)
    def _head_correction():
        qn = qn_ref[...]
        mem_head = mem_ref[0:T, :]
        dec_head = DECAY * mem_head + dmem_ref[...]
        decb = dec_head.astype(jnp.bfloat16)
        str2h = DECAY * str_ref[...][:, 0:T] + ss_ref[...]
        nssh = jax.lax.dot_general(
            ones_row, dec_head * dec_head, (((1,), (1,)), ((), ())),
            preferred_element_type=jnp.float32)
        rmh = str2h / jnp.maximum(jnp.sqrt(nssh), 1e-12)
        lt = jax.lax.dot_general(
            qn, decb, (((1,), (1,)), ((), ())),
            preferred_element_type=jnp.float32)
        pt = jnp.exp(lt * rmh)
        ptb = pt.astype(jnp.bfloat16)
        vdecb = (dec_head * (1.0 / DECAY)).astype(jnp.bfloat16)
        memb_head = memb[0:T]
        lw = jax.lax.dot_general(
            qn, memb_head, (((1,), (1,)), ((), ())),
            preferred_element_type=jnp.float32)
        pw = jnp.exp(lw * rowmul[:, 0:T])
        pwb = pw.astype(jnp.bfloat16)
        vdec_aug = jnp.concatenate(
            [vdecb, jnp.ones((T, 1), jnp.bfloat16)], axis=1)
        mem_head_aug = jnp.concatenate(
            [memb_head, jnp.ones((T, 1), jnp.bfloat16)], axis=1)
        acc_ref[...] += (
            jax.lax.dot_general(ptb, vdec_aug, (((1,), (0,)), ((), ())),
                                preferred_element_type=jnp.float32)
            - jax.lax.dot_general(pwb, mem_head_aug, (((1,), (0,)), ((), ())),
                                  preferred_element_type=jnp.float32))

    @pl.when(j == nb - 1)
    def _finalize():
        acc = acc_ref[...]
        out_ref[...] = (DECAY * acc[:, :D]) / acc[:, D:D + 1]


def kernel(h, surprise, mem, strength):
    tok, ss = _sc_sort(surprise)
    hf = h.reshape(QT, D)
    strr = strength.reshape(1, M)
    out = pl.pallas_call(
        _flash_body,
        grid=(M // BM,),
        in_specs=[
            pl.BlockSpec((QT, D), lambda j: (0, 0)),
            pl.BlockSpec((BM, D), lambda j: (j, 0)),
            pl.BlockSpec((1, BM), lambda j: (0, j)),
            pl.BlockSpec((T, 1), lambda j: (0, 0)),
            pl.BlockSpec((1, T), lambda j: (0, 0)),
        ],
        out_specs=pl.BlockSpec((QT, D), lambda j: (0, 0)),
        out_shape=jax.ShapeDtypeStruct((QT, D), jnp.float32),
        scratch_shapes=[
            pltpu.VMEM((QT, D), jnp.bfloat16),
            pltpu.VMEM((QT, D + 1), jnp.float32),
            pltpu.VMEM((T, D), jnp.float32),
        ],
        compiler_params=pltpu.CompilerParams(
            dimension_semantics=("arbitrary",)),
    )(hf, mem, strr, tok.reshape(T, 1), ss.reshape(1, T))
    return out.reshape(B, T, D)


# speedup vs baseline: 1.0364x; 1.0364x over previous
import functools

import jax
import jax.numpy as jnp
from jax import lax
from jax.experimental import pallas as pl
from jax.experimental.pallas import tpu as pltpu, tpu_sc as plsc

DECAY = 0.98
LR = 0.05
B, T, D = 4, 128, 64
M = 65536
BM = 4096
QT = B * T
L = 16
NV = T // L


def _sc_sort_body(sur_hbm, tok_hbm, ss_hbm, sur_v, tok_v, ss_v):
    wid = lax.axis_index("s") * 2 + lax.axis_index("c")

    @pl.when(wid == 0)
    def _tile0():
        pltpu.sync_copy(sur_hbm, sur_v)
        svecs = []
        for v in range(NV):
            acc = sur_v[0, pl.ds(v * L, L)]
            for b in range(1, B):
                acc = acc + sur_v[b, pl.ds(v * L, L)]
            sv = acc * 0.25
            svecs.append(sv)

        cnts = tuple(jnp.zeros((L,), jnp.int32) for _ in range(NV))
        for ub in range(NV):
            src = svecs[ub]

            def rank_step(r, cnts, ub=ub, src=src):
                su = lax.gather(
                    src, jnp.full((L, 1), r, jnp.int32),
                    dimension_numbers=lax.GatherDimensionNumbers(
                        offset_dims=(), collapsed_slice_dims=(0,),
                        start_index_map=(0,)),
                    slice_sizes=(1,),
                    mode=lax.GatherScatterMode.PROMISE_IN_BOUNDS)
                u = ub * L + r
                new = []
                for v in range(NV):
                    lane = lax.iota(jnp.int32, L) + (v * L)
                    gt = (su > svecs[v]).astype(jnp.int32)
                    tie = ((su == svecs[v]) & (u < lane)).astype(jnp.int32)
                    new.append(cnts[v] + gt + tie)
                return tuple(new)

            cnts = lax.fori_loop(0, L, rank_step, cnts)

        for v in range(NV):
            lane = lax.iota(jnp.int32, L) + (v * L)
            plsc.store_scatter(tok_v, [cnts[v]], lane)
            plsc.store_scatter(ss_v, [cnts[v]], svecs[v])
        pltpu.sync_copy(tok_v, tok_hbm)
        pltpu.sync_copy(ss_v, ss_hbm)


_sc_sort = functools.partial(
    pl.kernel,
    out_type=(jax.ShapeDtypeStruct((T,), jnp.int32),
              jax.ShapeDtypeStruct((T,), jnp.float32)),
    compiler_params=pltpu.CompilerParams(needs_layout_passes=False),
    mesh=plsc.VectorSubcoreMesh(core_axis_name="c", subcore_axis_name="s"),
    scratch_types=[
        pltpu.VMEM((B, T), jnp.float32),
        pltpu.VMEM((T,), jnp.int32),
        pltpu.VMEM((T,), jnp.float32),
    ],
)(_sc_sort_body)


def _flash_body(hf_ref, mem_ref, str_ref, tok_ref, ss_ref, out_ref,
                qn_ref, acc_ref, dmem_ref):
    j = pl.program_id(0)
    nb = pl.num_programs(0)
    ones_row = jnp.ones((1, D), jnp.float32)

    @pl.when(j == 0)
    def _prologue():
        hfv = hf_ref[...]
        qss = jnp.sum(hfv * hfv, axis=1, keepdims=True)
        qn_ref[...] = (hfv / jnp.maximum(jnp.sqrt(qss), 1e-12)
                       ).astype(jnp.bfloat16)

        c_io = jax.lax.broadcasted_iota(jnp.int32, (T, T), 1)
        ohot = (tok_ref[...] == c_io).astype(jnp.float32)

        mh = (hfv[0:T] + hfv[T:2 * T] + hfv[2 * T:3 * T] + hfv[3 * T:4 * T]) * 0.25
        mss = jnp.sum(mh * mh, axis=1, keepdims=True)
        mhn = mh / jnp.maximum(jnp.sqrt(mss), 1e-12)
        dmem_ref[...] = LR * jax.lax.dot_general(
            ohot, mhn, (((1,), (0,)), ((), ())),
            preferred_element_type=jnp.float32)

        acc_ref[...] = jnp.zeros((QT, D + 1), jnp.float32)

    mem_blk = mem_ref[...]
    memb = mem_blk.astype(jnp.bfloat16)
    sq = mem_blk * mem_blk
    nss = jax.lax.dot_general(
        ones_row, sq, (((1,), (1,)), ((), ())),
        preferred_element_type=jnp.float32)
    rowmul = (DECAY * DECAY * str_ref[...]) / jnp.maximum(
        DECAY * jnp.sqrt(nss), 1e-12)
    logits = jax.lax.dot_general(
        qn_ref[...], memb, (((1,), (1,)), ((), ())),
        preferred_element_type=jnp.float32)
    p = jnp.exp(logits * rowmul)
    pb = p.astype(jnp.bfloat16)
    vaug = jnp.concatenate(
        [memb, jnp.ones((BM, 1), jnp.bfloat16)], axis=1)
    acc_ref[...] += jax.lax.dot_general(
        pb, vaug, (((1,), (0,)), ((), ())),
        preferred_element_type=jnp.float32)

    @pl.when(j == 0)
    def _head_correction():
        qn = qn_ref[...]
        mem_head = mem_ref[0:T, :]
        dec_head = DECAY * mem_head + dmem_ref[...]
        decb = dec_head.astype(jnp.bfloat16)
        str2h = DECAY * str_ref[...][:, 0:T] + ss_ref[...]
        nssh = jax.lax.dot_general(
            ones_row, dec_head * dec_head, (((1,), (1,)), ((), ())),
            preferred_element_type=jnp.float32)
        rmh = str2h / jnp.maximum(jnp.sqrt(nssh), 1e-12)
        lt = jax.lax.dot_general(
            qn, decb, (((1,), (1,)), ((), ())),
            preferred_element_type=jnp.float32)
        pt = jnp.exp(lt * rmh)
        ptb = pt.astype(jnp.bfloat16)
        vdecb = (dec_head * (1.0 / DECAY)).astype(jnp.bfloat16)
        memb_head = memb[0:T]
        lw = jax.lax.dot_general(
            qn, memb_head, (((1,), (1,)), ((), ())),
            preferred_element_type=jnp.float32)
        pw = jnp.exp(lw * rowmul[:, 0:T])
        pwb = pw.astype(jnp.bfloat16)
        vdec_aug = jnp.concatenate(
            [vdecb, jnp.ones((T, 1), jnp.bfloat16)], axis=1)
        mem_head_aug = jnp.concatenate(
            [memb_head, jnp.ones((T, 1), jnp.bfloat16)], axis=1)
        acc_ref[...] += (
            jax.lax.dot_general(ptb, vdec_aug, (((1,), (0,)), ((), ())),
                                preferred_element_type=jnp.float32)
            - jax.lax.dot_general(pwb, mem_head_aug, (((1,), (0,)), ((), ())),
                                  preferred_element_type=jnp.float32))

    @pl.when(j == nb - 1)
    def _finalize():
        acc = acc_ref[...]
        out_ref[...] = (DECAY * acc[:, :D]) / acc[:, D:D + 1]


def kernel(h, surprise, mem, strength):
    tok, ss = _sc_sort(surprise)
    hf = h.reshape(QT, D)
    strr = strength.reshape(1, M)
    out = pl.pallas_call(
        _flash_body,
        grid=(M // BM,),
        in_specs=[
            pl.BlockSpec((QT, D), lambda j: (0, 0)),
            pl.BlockSpec((BM, D), lambda j: (j, 0)),
            pl.BlockSpec((1, BM), lambda j: (0, j)),
            pl.BlockSpec((T, 1), lambda j: (0, 0)),
            pl.BlockSpec((1, T), lambda j: (0, 0)),
        ],
        out_specs=pl.BlockSpec((QT, D), lambda j: (0, 0)),
        out_shape=jax.ShapeDtypeStruct((QT, D), jnp.float32),
        scratch_shapes=[
            pltpu.VMEM((QT, D), jnp.bfloat16),
            pltpu.VMEM((QT, D + 1), jnp.float32),
            pltpu.VMEM((T, D), jnp.float32),
        ],
        compiler_params=pltpu.CompilerParams(
            dimension_semantics=("arbitrary",)),
    )(hf, mem, strr, tok.reshape(T, 1), ss.reshape(1, T))
    return out.reshape(B, T, D)


# speedup vs baseline: 1.0444x; 1.0077x over previous
import functools

import jax
import jax.numpy as jnp
from jax import lax
from jax.experimental import pallas as pl
from jax.experimental.pallas import tpu as pltpu, tpu_sc as plsc

DECAY = 0.98
LR = 0.05
B, T, D = 4, 128, 64
M = 65536
BM = 8192
QT = B * T
L = 16
NV = T // L


def _sc_sort_body(sur_hbm, tok_hbm, ss_hbm, sur_v, tok_v, ss_v):
    wid = lax.axis_index("s") * 2 + lax.axis_index("c")

    @pl.when(wid == 0)
    def _tile0():
        pltpu.sync_copy(sur_hbm, sur_v)
        svecs = []
        for v in range(NV):
            acc = sur_v[0, pl.ds(v * L, L)]
            for b in range(1, B):
                acc = acc + sur_v[b, pl.ds(v * L, L)]
            sv = acc * 0.25
            svecs.append(sv)

        cnts = tuple(jnp.zeros((L,), jnp.int32) for _ in range(NV))
        for ub in range(NV):
            src = svecs[ub]

            def rank_step(r, cnts, ub=ub, src=src):
                su = lax.gather(
                    src, jnp.full((L, 1), r, jnp.int32),
                    dimension_numbers=lax.GatherDimensionNumbers(
                        offset_dims=(), collapsed_slice_dims=(0,),
                        start_index_map=(0,)),
                    slice_sizes=(1,),
                    mode=lax.GatherScatterMode.PROMISE_IN_BOUNDS)
                u = ub * L + r
                new = []
                for v in range(NV):
                    lane = lax.iota(jnp.int32, L) + (v * L)
                    gt = (su > svecs[v]).astype(jnp.int32)
                    tie = ((su == svecs[v]) & (u < lane)).astype(jnp.int32)
                    new.append(cnts[v] + gt + tie)
                return tuple(new)

            cnts = lax.fori_loop(0, L, rank_step, cnts)

        for v in range(NV):
            lane = lax.iota(jnp.int32, L) + (v * L)
            plsc.store_scatter(tok_v, [cnts[v]], lane)
            plsc.store_scatter(ss_v, [cnts[v]], svecs[v])
        pltpu.sync_copy(tok_v, tok_hbm)
        pltpu.sync_copy(ss_v, ss_hbm)


_sc_sort = functools.partial(
    pl.kernel,
    out_type=(jax.ShapeDtypeStruct((T,), jnp.int32),
              jax.ShapeDtypeStruct((T,), jnp.float32)),
    compiler_params=pltpu.CompilerParams(needs_layout_passes=False),
    mesh=plsc.VectorSubcoreMesh(core_axis_name="c", subcore_axis_name="s"),
    scratch_types=[
        pltpu.VMEM((B, T), jnp.float32),
        pltpu.VMEM((T,), jnp.int32),
        pltpu.VMEM((T,), jnp.float32),
    ],
)(_sc_sort_body)


def _flash_body(hf_ref, mem_ref, str_ref, tok_ref, ss_ref, out_ref,
                qn_ref, acc_ref, dmem_ref):
    j = pl.program_id(0)
    nb = pl.num_programs(0)
    ones_row = jnp.ones((1, D), jnp.float32)

    @pl.when(j == 0)
    def _prologue():
        hfv = hf_ref[...]
        qss = jnp.sum(hfv * hfv, axis=1, keepdims=True)
        qn_ref[...] = (hfv / jnp.maximum(jnp.sqrt(qss), 1e-12)
                       ).astype(jnp.bfloat16)

        c_io = jax.lax.broadcasted_iota(jnp.int32, (T, T), 1)
        ohot = (tok_ref[...] == c_io).astype(jnp.float32)

        mh = (hfv[0:T] + hfv[T:2 * T] + hfv[2 * T:3 * T] + hfv[3 * T:4 * T]) * 0.25
        mss = jnp.sum(mh * mh, axis=1, keepdims=True)
        mhn = mh / jnp.maximum(jnp.sqrt(mss), 1e-12)
        dmem_ref[...] = LR * jax.lax.dot_general(
            ohot, mhn, (((1,), (0,)), ((), ())),
            preferred_element_type=jnp.float32)

        acc_ref[...] = jnp.zeros((QT, D + 1), jnp.float32)

    mem_blk = mem_ref[...]
    memb = mem_blk.astype(jnp.bfloat16)
    sq = mem_blk * mem_blk
    nss = jax.lax.dot_general(
        ones_row, sq, (((1,), (1,)), ((), ())),
        preferred_element_type=jnp.float32)
    rowmul = (DECAY * DECAY * str_ref[...]) / jnp.maximum(
        DECAY * jnp.sqrt(nss), 1e-12)
    logits = jax.lax.dot_general(
        qn_ref[...], memb, (((1,), (1,)), ((), ())),
        preferred_element_type=jnp.float32)
    p = jnp.exp(logits * rowmul)
    pb = p.astype(jnp.bfloat16)
    vaug = jnp.concatenate(
        [memb, jnp.ones((BM, 1), jnp.bfloat16)], axis=1)
    acc_ref[...] += jax.lax.dot_general(
        pb, vaug, (((1,), (0,)), ((), ())),
        preferred_element_type=jnp.float32)

    @pl.when(j == 0)
    def _head_correction():
        qn = qn_ref[...]
        mem_head = mem_ref[0:T, :]
        dec_head = DECAY * mem_head + dmem_ref[...]
        decb = dec_head.astype(jnp.bfloat16)
        str2h = DECAY * str_ref[...][:, 0:T] + ss_ref[...]
        nssh = jax.lax.dot_general(
            ones_row, dec_head * dec_head, (((1,), (1,)), ((), ())),
            preferred_element_type=jnp.float32)
        rmh = str2h / jnp.maximum(jnp.sqrt(nssh), 1e-12)
        lt = jax.lax.dot_general(
            qn, decb, (((1,), (1,)), ((), ())),
            preferred_element_type=jnp.float32)
        pt = jnp.exp(lt * rmh)
        ptb = pt.astype(jnp.bfloat16)
        vdecb = (dec_head * (1.0 / DECAY)).astype(jnp.bfloat16)
        memb_head = memb[0:T]
        lw = jax.lax.dot_general(
            qn, memb_head, (((1,), (1,)), ((), ())),
            preferred_element_type=jnp.float32)
        pw = jnp.exp(lw * rowmul[:, 0:T])
        pwb = pw.astype(jnp.bfloat16)
        vdec_aug = jnp.concatenate(
            [vdecb, jnp.ones((T, 1), jnp.bfloat16)], axis=1)
        mem_head_aug = jnp.concatenate(
            [memb_head, jnp.ones((T, 1), jnp.bfloat16)], axis=1)
        acc_ref[...] += (
            jax.lax.dot_general(ptb, vdec_aug, (((1,), (0,)), ((), ())),
                                preferred_element_type=jnp.float32)
            - jax.lax.dot_general(pwb, mem_head_aug, (((1,), (0,)), ((), ())),
                                  preferred_element_type=jnp.float32))

    @pl.when(j == nb - 1)
    def _finalize():
        acc = acc_ref[...]
        out_ref[...] = (DECAY * acc[:, :D]) / acc[:, D:D + 1]


def kernel(h, surprise, mem, strength):
    tok, ss = _sc_sort(surprise)
    hf = h.reshape(QT, D)
    strr = strength.reshape(1, M)
    out = pl.pallas_call(
        _flash_body,
        grid=(M // BM,),
        in_specs=[
            pl.BlockSpec((QT, D), lambda j: (0, 0)),
            pl.BlockSpec((BM, D), lambda j: (j, 0)),
            pl.BlockSpec((1, BM), lambda j: (0, j)),
            pl.BlockSpec((T, 1), lambda j: (0, 0)),
            pl.BlockSpec((1, T), lambda j: (0, 0)),
        ],
        out_specs=pl.BlockSpec((QT, D), lambda j: (0, 0)),
        out_shape=jax.ShapeDtypeStruct((QT, D), jnp.float32),
        scratch_shapes=[
            pltpu.VMEM((QT, D), jnp.bfloat16),
            pltpu.VMEM((QT, D + 1), jnp.float32),
            pltpu.VMEM((T, D), jnp.float32),
        ],
        compiler_params=pltpu.CompilerParams(
            dimension_semantics=("arbitrary",)),
    )(hf, mem, strr, tok.reshape(T, 1), ss.reshape(1, T))
    return out.reshape(B, T, D)


# speedup vs baseline: 1.0480x; 1.0034x over previous
import functools

import jax
import jax.numpy as jnp
from jax import lax
from jax.experimental import pallas as pl
from jax.experimental.pallas import tpu as pltpu, tpu_sc as plsc

DECAY = 0.98
LR = 0.05
B, T, D = 4, 128, 64
M = 65536
BM = 8192
QT = B * T
L = 16
NV = T // L


def _sc_sort_body(sur_hbm, tok_hbm, ss_hbm, sur_v, tok_v, ss_v):
    wid = lax.axis_index("s") * 2 + lax.axis_index("c")

    @pl.when(wid == 0)
    def _tile0():
        pltpu.sync_copy(sur_hbm, sur_v)
        svecs = []
        for v in range(NV):
            acc = sur_v[0, pl.ds(v * L, L)]
            for b in range(1, B):
                acc = acc + sur_v[b, pl.ds(v * L, L)]
            sv = acc * 0.25
            svecs.append(sv)

        cnts = tuple(jnp.zeros((L,), jnp.int32) for _ in range(NV))
        for ub in range(NV):
            src = svecs[ub]

            def rank_step(r, cnts, ub=ub, src=src):
                su = lax.gather(
                    src, jnp.full((L, 1), r, jnp.int32),
                    dimension_numbers=lax.GatherDimensionNumbers(
                        offset_dims=(), collapsed_slice_dims=(0,),
                        start_index_map=(0,)),
                    slice_sizes=(1,),
                    mode=lax.GatherScatterMode.PROMISE_IN_BOUNDS)
                u = ub * L + r
                new = []
                for v in range(NV):
                    lane = lax.iota(jnp.int32, L) + (v * L)
                    gt = (su > svecs[v]).astype(jnp.int32)
                    tie = ((su == svecs[v]) & (u < lane)).astype(jnp.int32)
                    new.append(cnts[v] + gt + tie)
                return tuple(new)

            cnts = lax.fori_loop(0, L, rank_step, cnts)

        for v in range(NV):
            lane = lax.iota(jnp.int32, L) + (v * L)
            plsc.store_scatter(tok_v, [cnts[v]], lane)
            plsc.store_scatter(ss_v, [cnts[v]], svecs[v])
        pltpu.sync_copy(tok_v, tok_hbm)
        pltpu.sync_copy(ss_v, ss_hbm)


_sc_sort = functools.partial(
    pl.kernel,
    out_type=(jax.ShapeDtypeStruct((T,), jnp.int32),
              jax.ShapeDtypeStruct((T,), jnp.float32)),
    compiler_params=pltpu.CompilerParams(needs_layout_passes=False),
    mesh=plsc.VectorSubcoreMesh(core_axis_name="c", subcore_axis_name="s"),
    scratch_types=[
        pltpu.VMEM((B, T), jnp.float32),
        pltpu.VMEM((T,), jnp.int32),
        pltpu.VMEM((T,), jnp.float32),
    ],
)(_sc_sort_body)


def _flash_body(hf_ref, mem_ref, str_ref, acc_out_ref, qn_ref, acc_ref):
    j = pl.program_id(0)
    nb = pl.num_programs(0)
    ones_row = jnp.ones((1, D), jnp.float32)

    @pl.when(j == 0)
    def _prologue():
        hfv = hf_ref[...]
        qss = jnp.sum(hfv * hfv, axis=1, keepdims=True)
        qn_ref[...] = (hfv / jnp.maximum(jnp.sqrt(qss), 1e-12)
                       ).astype(jnp.bfloat16)
        acc_ref[...] = jnp.zeros((QT, D + 1), jnp.float32)

    mem_blk = mem_ref[...]
    memb = mem_blk.astype(jnp.bfloat16)
    sq = mem_blk * mem_blk
    nss = jax.lax.dot_general(
        ones_row, sq, (((1,), (1,)), ((), ())),
        preferred_element_type=jnp.float32)
    rowmul = (DECAY * DECAY * str_ref[...]) / jnp.maximum(
        DECAY * jnp.sqrt(nss), 1e-12)
    logits = jax.lax.dot_general(
        qn_ref[...], memb, (((1,), (1,)), ((), ())),
        preferred_element_type=jnp.float32)
    p = jnp.exp(logits * rowmul)
    pb = p.astype(jnp.bfloat16)
    vaug = jnp.concatenate(
        [memb, jnp.ones((BM, 1), jnp.bfloat16)], axis=1)
    acc_ref[...] += jax.lax.dot_general(
        pb, vaug, (((1,), (0,)), ((), ())),
        preferred_element_type=jnp.float32)

    @pl.when(j == nb - 1)
    def _store():
        acc_out_ref[...] = acc_ref[...]


def _fix_body(acc_ref, hf_ref, memh_ref, strh_ref, tok_ref, ss_ref, out_ref):
    ones_row = jnp.ones((1, D), jnp.float32)
    hfv = hf_ref[...]
    qss = jnp.sum(hfv * hfv, axis=1, keepdims=True)
    qn = (hfv / jnp.maximum(jnp.sqrt(qss), 1e-12)).astype(jnp.bfloat16)

    c_io = jax.lax.broadcasted_iota(jnp.int32, (T, T), 1)
    ohot = (tok_ref[...] == c_io).astype(jnp.float32)
    mh = (hfv[0:T] + hfv[T:2 * T] + hfv[2 * T:3 * T] + hfv[3 * T:4 * T]) * 0.25
    mss = jnp.sum(mh * mh, axis=1, keepdims=True)
    mhn = mh / jnp.maximum(jnp.sqrt(mss), 1e-12)
    delta = LR * jax.lax.dot_general(
        ohot, mhn, (((1,), (0,)), ((), ())),
        preferred_element_type=jnp.float32)

    mem_head = memh_ref[...]
    strh = strh_ref[...]
    membh = mem_head.astype(jnp.bfloat16)
    sqh = mem_head * mem_head
    nssw = jax.lax.dot_general(
        ones_row, sqh, (((1,), (1,)), ((), ())),
        preferred_element_type=jnp.float32)
    rowmul_w = (DECAY * DECAY * strh) / jnp.maximum(
        DECAY * jnp.sqrt(nssw), 1e-12)
    lw = jax.lax.dot_general(
        qn, membh, (((1,), (1,)), ((), ())),
        preferred_element_type=jnp.float32)
    pw = jnp.exp(lw * rowmul_w)
    pwb = pw.astype(jnp.bfloat16)

    dec_head = DECAY * mem_head + delta
    decb = dec_head.astype(jnp.bfloat16)
    str2h = DECAY * strh + ss_ref[...]
    nssh = jax.lax.dot_general(
        ones_row, dec_head * dec_head, (((1,), (1,)), ((), ())),
        preferred_element_type=jnp.float32)
    rmh = str2h / jnp.maximum(jnp.sqrt(nssh), 1e-12)
    lt = jax.lax.dot_general(
        qn, decb, (((1,), (1,)), ((), ())),
        preferred_element_type=jnp.float32)
    pt = jnp.exp(lt * rmh)
    ptb = pt.astype(jnp.bfloat16)
    vdecb = (dec_head * (1.0 / DECAY)).astype(jnp.bfloat16)

    vdec_aug = jnp.concatenate(
        [vdecb, jnp.ones((T, 1), jnp.bfloat16)], axis=1)
    memh_aug = jnp.concatenate(
        [membh, jnp.ones((T, 1), jnp.bfloat16)], axis=1)
    acc = acc_ref[...] + (
        jax.lax.dot_general(ptb, vdec_aug, (((1,), (0,)), ((), ())),
                            preferred_element_type=jnp.float32)
        - jax.lax.dot_general(pwb, memh_aug, (((1,), (0,)), ((), ())),
                              preferred_element_type=jnp.float32))
    out_ref[...] = (DECAY * acc[:, :D]) / acc[:, D:D + 1]


def kernel(h, surprise, mem, strength):
    tok, ss = _sc_sort(surprise)
    hf = h.reshape(QT, D)
    strr = strength.reshape(1, M)
    acc = pl.pallas_call(
        _flash_body,
        grid=(M // BM,),
        in_specs=[
            pl.BlockSpec((QT, D), lambda j: (0, 0)),
            pl.BlockSpec((BM, D), lambda j: (j, 0)),
            pl.BlockSpec((1, BM), lambda j: (0, j)),
        ],
        out_specs=pl.BlockSpec((QT, D + 1), lambda j: (0, 0)),
        out_shape=jax.ShapeDtypeStruct((QT, D + 1), jnp.float32),
        scratch_shapes=[
            pltpu.VMEM((QT, D), jnp.bfloat16),
            pltpu.VMEM((QT, D + 1), jnp.float32),
        ],
        compiler_params=pltpu.CompilerParams(
            dimension_semantics=("arbitrary",)),
    )(hf, mem, strr)
    out = pl.pallas_call(
        _fix_body,
        grid=(1,),
        in_specs=[
            pl.BlockSpec((QT, D + 1), lambda j: (0, 0)),
            pl.BlockSpec((QT, D), lambda j: (0, 0)),
            pl.BlockSpec((T, D), lambda j: (0, 0)),
            pl.BlockSpec((1, T), lambda j: (0, 0)),
            pl.BlockSpec((T, 1), lambda j: (0, 0)),
            pl.BlockSpec((1, T), lambda j: (0, 0)),
        ],
        out_specs=pl.BlockSpec((QT, D), lambda j: (0, 0)),
        out_shape=jax.ShapeDtypeStruct((QT, D), jnp.float32),
    )(acc, hf, mem, strr, tok.reshape(T, 1), ss.reshape(1, T))
    return out.reshape(B, T, D)
